# baseline (speedup 1.0000x reference)
import functools

import jax
import jax.numpy as jnp
import numpy as np
from jax import lax
from jax.experimental import pallas as pl
from jax.experimental.pallas import tpu as pltpu
from jax.experimental.pallas import tpu_sc as plsc

NTILES = 16
NCORES = 2
KE = 128

f32 = jnp.float32
i32 = jnp.int32


def _rup(x, m):
    return -(-x // m) * m


def _mesh():
    return plsc.VectorSubcoreMesh(core_axis_name="c", subcore_axis_name="s")


def _pad_edges(src, dst, trash):
    E = src.shape[0]
    NB = -(-E // (NTILES * KE))
    pad = NTILES * KE * NB - E
    srcp = jnp.concatenate([src, jnp.zeros((pad,), i32)])
    dstp = jnp.concatenate([dst, jnp.full((pad,), trash, i32)])
    return srcp, dstp, NB


def _seq_dst_pad(dst, trash):
    E = dst.shape[0]
    sh8 = _rup(-(-E // NTILES), 8)
    NB = -(-sh8 // KE)
    covered = np.zeros(E, bool)
    rowidx = np.zeros((NTILES, NB, KE), np.int32)
    newmask = np.zeros((NTILES, NB, KE), bool)
    for t in range(NTILES):
        for j in range(NB):
            s = min(t * sh8 + min(j * KE, sh8 - KE), E - KE)
            rows = np.arange(s, s + KE)
            rowidx[t, j] = rows
            newmask[t, j] = ~covered[rows]
            covered[rows] = True
    assert covered.all()
    dstp = jnp.where(jnp.asarray(newmask), dst[jnp.asarray(rowidx)], trash)
    return dstp.reshape(-1), NB, sh8, E


def _sc_gather_scatter(NB, C, CW, N, Np):
    CPC = C // NCORES
    rows = Np // NTILES
    shard = NB * KE

    @functools.partial(
        pl.kernel,
        mesh=_mesh(),
        compiler_params=pltpu.CompilerParams(use_tc_tiling_on_sc=False),
        out_type=jax.ShapeDtypeStruct((C, Np, CW), f32),
        scratch_types=[
            pltpu.VMEM((KE,), i32),
            pltpu.VMEM((KE,), i32),
            pltpu.VMEM((KE, CW), f32),
            pltpu.VMEM_SHARED((Np, CW), f32),
            pltpu.SemaphoreType.DMA,
        ],
    )
    def k(table, srcp, dstp, zeros, out, idx_s, idx_d, vals, acc, sem):
        cid = lax.axis_index("c")
        sid = lax.axis_index("s")
        r0 = sid * rows
        for cc in range(CPC):
            chunk = cid * CPC + cc
            pltpu.sync_copy(zeros.at[pl.ds(r0, rows)], acc.at[pl.ds(r0, rows)])
            plsc.subcore_barrier()

            def blk(j, carry):
                off = sid * shard + j * KE
                pltpu.sync_copy(srcp.at[pl.ds(off, KE)], idx_s)
                pltpu.sync_copy(dstp.at[pl.ds(off, KE)], idx_d)
                pltpu.async_copy(table.at[chunk].at[idx_s], vals, sem).wait()
                pltpu.sync_copy(vals, acc.at[idx_d], add=True)
                return carry

            lax.fori_loop(0, NB, blk, 0)
            plsc.subcore_barrier()
            pltpu.sync_copy(acc.at[pl.ds(r0, rows)],
                            out.at[chunk].at[pl.ds(r0, rows)])

    return k


def _sc_seq_scatter(NB, sh8, E, C, CW, Np, chunk_major):
    CPC = C // NCORES
    rows = Np // NTILES

    @functools.partial(
        pl.kernel,
        mesh=_mesh(),
        compiler_params=pltpu.CompilerParams(use_tc_tiling_on_sc=False),
        out_type=jax.ShapeDtypeStruct((C, Np, CW), f32),
        scratch_types=[
            pltpu.VMEM((KE,), i32),
            pltpu.VMEM((KE, CW), f32),
            pltpu.VMEM_SHARED((Np, CW), f32),
        ],
    )
    def k(ef, dstp, zeros, out, idx_d, vals, acc):
        cid = lax.axis_index("c")
        sid = lax.axis_index("s")
        r0 = sid * rows
        for cc in range(CPC):
            chunk = cid * CPC + cc
            pltpu.sync_copy(zeros.at[pl.ds(r0, rows)], acc.at[pl.ds(r0, rows)])
            plsc.subcore_barrier()

            def blk(j, carry):
                off_v = jnp.minimum(
                    sid * sh8 + jnp.minimum(j * KE, sh8 - KE), E - KE)
                pltpu.sync_copy(dstp.at[pl.ds((sid * NB + j) * KE, KE)], idx_d)
                if chunk_major:
                    pltpu.sync_copy(ef.at[chunk].at[pl.ds(off_v, KE)], vals)
                else:
                    pltpu.sync_copy(
                        ef.at[pl.ds(off_v, KE), pl.ds(chunk * CW, CW)], vals)
                pltpu.sync_copy(vals, acc.at[idx_d], add=True)
                return carry

            lax.fori_loop(0, NB, blk, 0)
            plsc.subcore_barrier()
            pltpu.sync_copy(acc.at[pl.ds(r0, rows)],
                            out.at[chunk].at[pl.ds(r0, rows)])

    return k


def _sc_counts(jobs):
    out_types = [jax.ShapeDtypeStruct((Np, 16), f32) for (_, Np) in jobs]
    accN = max(Np for (_, Np) in jobs)

    @functools.partial(
        pl.kernel,
        mesh=_mesh(),
        compiler_params=pltpu.CompilerParams(use_tc_tiling_on_sc=False),
        out_type=out_types,
        scratch_types=[
            pltpu.VMEM((KE,), i32),
            pltpu.VMEM((KE, 16), f32),
            pltpu.VMEM_SHARED((accN, 16), f32),
        ],
    )
    def k(*args):
        nj = len(jobs)
        dstps = args[:nj]
        ones, zeros = args[nj], args[nj + 1]
        outs = args[nj + 2:nj + 2 + nj]
        idx_d, ones_v, acc = args[nj + 2 + nj:]
        cid = lax.axis_index("c")
        sid = lax.axis_index("s")
        pltpu.sync_copy(ones, ones_v)
        for jj, (NB, Np) in enumerate(jobs):
            rows = Np // NTILES
            shard = NB * KE

            @pl.when(cid == (jj % 2))
            def _job(jj=jj, NB=NB, rows=rows, shard=shard):
                r0 = sid * rows
                pltpu.sync_copy(zeros.at[pl.ds(r0, rows)],
                                acc.at[pl.ds(r0, rows)])
                plsc.subcore_barrier()

                def blk(j, carry):
                    off = sid * shard + j * KE
                    pltpu.sync_copy(dstps[jj].at[pl.ds(off, KE)], idx_d)
                    pltpu.sync_copy(ones_v, acc.at[idx_d], add=True)
                    return carry

                lax.fori_loop(0, NB, blk, 0)
                plsc.subcore_barrier()
                pltpu.sync_copy(acc.at[pl.ds(r0, rows)],
                                outs[jj].at[pl.ds(r0, rows)])

    return k


def _ln(x, g, b, eps=1e-5):
    m = jnp.mean(x, -1, keepdims=True)
    v = jnp.mean((x - m) ** 2, -1, keepdims=True)
    return (x - m) * lax.rsqrt(v + eps) * g + b


def _row(BN, D):
    return pl.BlockSpec((BN, D), lambda i: (i, 0))


def _full(shape):
    return pl.BlockSpec(shape, lambda i: tuple(0 for _ in shape))


def _chunked(C, BN, CW):
    return pl.BlockSpec((C, BN, CW), lambda i: (0, i, 0))


def _cat(ref):
    C = ref.shape[0]
    return jnp.concatenate([ref[c] for c in range(C)], axis=-1)


def _store(o_ref, y, out_c):
    if out_c is None:
        o_ref[...] = y
    else:
        C, CW = out_c
        for c in range(C):
            o_ref[c] = y[:, c * CW:(c + 1) * CW]


def _proj(x, W, b, relu=False, out_c=None, BN=1000):
    N, Kd = x.shape
    H = W.shape[1]

    def f(x_ref, w_ref, b_ref, o_ref):
        y = jnp.dot(x_ref[...], w_ref[...], preferred_element_type=f32)
        y = y + b_ref[...]
        if relu:
            y = jnp.maximum(y, 0.0)
        _store(o_ref, y, out_c)

    if out_c is None:
        out_shape = jax.ShapeDtypeStruct((N, H), f32)
        out_spec = _row(BN, H)
    else:
        C, CW = out_c
        out_shape = jax.ShapeDtypeStruct((C, N, CW), f32)
        out_spec = _chunked(C, BN, CW)
    return pl.pallas_call(
        f, grid=(N // BN,),
        in_specs=[_row(BN, Kd), _full(W.shape), _full((1, H))],
        out_specs=out_spec, out_shape=out_shape,
    )(x, W, b.reshape(1, H))


def _ctx_final(S, cu, cs, com, pub, Wepu, bepu, N, BN=1000):
    H = Wepu.shape[1]
    CS, CWS = S.shape[0], S.shape[2]

    def f(s_ref, cu_ref, cs_ref, com_ref, pub_ref, w_ref, b_ref, o_ref):
        cu1 = cu_ref[...][:, :1]
        cs1 = cs_ref[...][:, :1]
        y = jnp.dot(_cat(s_ref), w_ref[...], preferred_element_type=f32)
        y = y + cu1 * b_ref[...] + 0.3 * _cat(com_ref)
        y = jnp.where(cs1 > 0, y / jnp.maximum(cs1, 1.0), y)
        o_ref[...] = y + 0.3 * _cat(pub_ref)

    return pl.pallas_call(
        f, grid=(N // BN,),
        in_specs=[_chunked(CS, BN, CWS), _row(BN, 16), _row(BN, 16),
                  _chunked(8, BN, 32), _chunked(8, BN, 32),
                  _full(Wepu.shape), _full((1, H))],
        out_specs=_row(BN, H),
        out_shape=jax.ShapeDtypeStruct((N, H), f32),
    )(S, cu, cs, com, pub, Wepu, bepu.reshape(1, H))


def _ecom_proj(S, cd, Wepc, bepc, N, BN=2000):
    H = Wepc.shape[1]
    CS, CWS = S.shape[0], S.shape[2]

    def f(s_ref, c_ref, w_ref, b_ref, o_ref):
        c1 = c_ref[...][:, :1]
        o_ref[...] = (jnp.dot(_cat(s_ref), w_ref[...], preferred_element_type=f32)
                      + c1 * b_ref[...])

    return pl.pallas_call(
        f, grid=(N // BN,),
        in_specs=[_chunked(CS, BN, CWS), _row(BN, 16), _full(Wepc.shape),
                  _full((1, H))],
        out_specs=_row(BN, H),
        out_shape=jax.ShapeDtypeStruct((N, H), f32),
    )(S, cd, Wepc, bepc.reshape(1, H))


def _u_msg(h, ctx, Wa, Wb, b, g, bg, out_c=(8, 32), BN=1000):
    N, H = h.shape

    def f(h_ref, c_ref, wa_ref, wb_ref, b_ref, g_ref, bg_ref, o_ref):
        y = (jnp.dot(h_ref[...], wa_ref[...], preferred_element_type=f32)
             + jnp.dot(c_ref[...], wb_ref[...], preferred_element_type=f32)
             + b_ref[...])
        y = jnp.maximum(_ln(y, g_ref[...], bg_ref[...]), 0.0)
        _store(o_ref, y, out_c)

    C, CW = out_c
    return pl.pallas_call(
        f, grid=(N // BN,),
        in_specs=[_row(BN, H), _row(BN, H), _full(Wa.shape), _full(Wb.shape),
                  _full((1, H)), _full((1, H)), _full((1, H))],
        out_specs=_chunked(C, BN, CW),
        out_shape=jax.ShapeDtypeStruct((C, N, CW), f32),
    )(h, ctx, Wa, Wb, b.reshape(1, H), g.reshape(1, H), bg.reshape(1, H))


def _post_combine(h, pub_sum, com_sum, Ecom, cp, cc, Wec, bec, g, b, BN=2000):
    N, H = h.shape

    def f(h_ref, ps_ref, cs_ref, e_ref, cp_ref, cc_ref, w_ref, be_ref,
          g_ref, b_ref, o_ref):
        cp1 = jnp.maximum(cp_ref[...][:, :1], 1.0)
        ccr = cc_ref[...][:, :1]
        cc1 = jnp.maximum(ccr, 1.0)
        ep = (jnp.dot(e_ref[...], w_ref[...], preferred_element_type=f32)
              + ccr * be_ref[...])
        aggc = (0.7 * _cat(cs_ref) + 0.3 * ep) / cc1
        aggp = _cat(ps_ref) / cp1
        o_ref[...] = _ln(h_ref[...] + 0.5 * (aggp + aggc), g_ref[...], b_ref[...])

    return pl.pallas_call(
        f, grid=(N // BN,),
        in_specs=[_row(BN, H), _chunked(2, BN, 128), _chunked(2, BN, 128),
                  _row(BN, H), _row(BN, 16), _row(BN, 16), _full(Wec.shape),
                  _full((1, H)), _full((1, H)), _full((1, H))],
        out_specs=_row(BN, H),
        out_shape=jax.ShapeDtypeStruct((N, H), f32),
    )(h, pub_sum, com_sum, Ecom, cp, cc, Wec, bec.reshape(1, H),
      g.reshape(1, H), b.reshape(1, H))


def _user_combine(h, u_sum, cu, g, b, BN=2000):
    N, H = h.shape

    def f(h_ref, u_ref, c_ref, g_ref, b_ref, o_ref):
        c1 = jnp.maximum(c_ref[...][:, :1], 1.0)
        o_ref[...] = _ln(h_ref[...] + _cat(u_ref) / c1, g_ref[...], b_ref[...])

    return pl.pallas_call(
        f, grid=(N // BN,),
        in_specs=[_row(BN, H), _chunked(8, BN, 32), _row(BN, 16),
                  _full((1, H)), _full((1, H))],
        out_specs=_row(BN, H),
        out_shape=jax.ShapeDtypeStruct((N, H), f32),
    )(h, u_sum, cu, g.reshape(1, H), b.reshape(1, H))


def _classifier(h, W1, b1, g, b, W2p, b2p, BN=2000):
    N, H = h.shape
    Ho = W2p.shape[1]

    def f(h_ref, w1_ref, b1_ref, g_ref, b_ref, w2_ref, b2_ref, o_ref):
        y = jnp.dot(h_ref[...], w1_ref[...], preferred_element_type=f32) + b1_ref[...]
        y = jnp.maximum(_ln(y, g_ref[...], b_ref[...]), 0.0)
        o_ref[...] = jnp.dot(y, w2_ref[...], preferred_element_type=f32) + b2_ref[...]

    return pl.pallas_call(
        f, grid=(N // BN,),
        in_specs=[_row(BN, H), _full(W1.shape), _full((1, H)), _full((1, H)),
                  _full((1, H)), _full(W2p.shape), _full((1, Ho))],
        out_specs=_row(BN, Ho),
        out_shape=jax.ShapeDtypeStruct((N, Ho), f32),
    )(h, W1, b1.reshape(1, H), g.reshape(1, H), b.reshape(1, H), W2p,
      b2p.reshape(1, Ho))


def kernel(user_x, post_x, ef_com, ef_ucu, edge_pub, edge_com, edge_ucu,
           Wu, bu, Wp, bp, Wepc, bepc, Wepu, bepu, Wuce, buce,
           Wpub, bpub, Wcom, bcom, Wconv, bconv, gconv, bgconv, Wec, bec,
           g_ln_u, b_ln_u, g_ln_p, b_ln_p, W1, b1, gcls, bcls, W2, b2):
    NU, NPOST, H = user_x.shape[0], post_x.shape[0], Wu.shape[1]
    NUp, NPp = _rup(NU + 1, 128), _rup(NPOST + 1, 128)

    zU32 = jnp.zeros((NUp, 32), f32)
    zP128 = jnp.zeros((NPp, 128), f32)
    zU16 = jnp.zeros((NUp, 16), f32)
    ones16 = jnp.ones((KE, 16), f32)

    ucu_s, ucu_d, NB_ucu = _pad_edges(edge_ucu[0], edge_ucu[1], NU)
    com_s, com_d, NB_com = _pad_edges(edge_com[0], edge_com[1], NPOST)
    pub_s, pub_d, NB_pub = _pad_edges(edge_pub[0], edge_pub[1], NPOST)
    comR_s, comR_d, _ = _pad_edges(edge_com[1], edge_com[0], NU)
    pubR_s, pubR_d, _ = _pad_edges(edge_pub[1], edge_pub[0], NU)
    _, ucu_src_cnt, _ = _pad_edges(edge_ucu[0], edge_ucu[0], NU)
    _, com_src_cnt, _ = _pad_edges(edge_com[0], edge_com[0], NU)
    ucu_seq_d, NBs_ucu, sh_ucu, EU = _seq_dst_pad(edge_ucu[0], NU)
    com_seq_d, NBs_com, sh_com, EC = _seq_dst_pad(edge_com[1], NPOST)

    cnt_jobs = [(NB_ucu, NUp), (NB_ucu, NUp), (NB_com, NUp),
                (NB_com, NPp), (NB_pub, NPp)]
    c_ucu_src, c_ucu_dst, c_com_src, c_com_dst, c_pub_dst = _sc_counts(cnt_jobs)(
        ucu_src_cnt, ucu_d, com_src_cnt, com_d, pub_d, ones16, zU16)

    h_user = _proj(user_x, Wu, bu)
    h_post = _proj(post_x, Wp, bp, BN=2000)
    post_ctx = _proj(post_x, Wuce, buce, relu=True, out_c=(8, 32), BN=2000)

    IN = ef_ucu.shape[1]
    ef_ucu_cm = jnp.transpose(ef_ucu.reshape(EU, IN // 32, 32), (1, 0, 2))
    S_ucu = _sc_seq_scatter(NBs_ucu, sh_ucu, EU, IN // 32, 32, NUp, True)(
        ef_ucu_cm, ucu_seq_d, zU32)
    S_com = _sc_seq_scatter(NBs_com, sh_com, EC, IN // 128, 128, NPp, False)(
        ef_com, com_seq_d, zP128)

    ctx_com = _sc_gather_scatter(NB_com, 8, 32, NU, NUp)(
        post_ctx, comR_s, comR_d, zU32)
    ctx_pub = _sc_gather_scatter(NB_pub, 8, 32, NU, NUp)(
        post_ctx, pubR_s, pubR_d, zU32)
    ctx = _ctx_final(S_ucu, c_ucu_src, c_com_src, ctx_com, ctx_pub,
                     Wepu, bepu, NU)
    Ecom = _ecom_proj(S_com, c_com_dst, Wepc, bepc, NPOST)

    gs_pub = _sc_gather_scatter(NB_pub, 2, 128, NPOST, NPp)
    gs_com = _sc_gather_scatter(NB_com, 2, 128, NPOST, NPp)
    gs_ucu = _sc_gather_scatter(NB_ucu, 8, 32, NU, NUp)

    L = Wpub.shape[0]
    for i in range(L):
        t_pub = _proj(h_user, Wpub[i], bpub[i], out_c=(2, 128))
        t_com = _proj(h_user, Wcom[i], bcom[i], out_c=(2, 128))
        pub_sum = gs_pub(t_pub, pub_s, pub_d, zP128)
        com_sum = gs_com(t_com, com_s, com_d, zP128)
        U = _u_msg(h_user, ctx, Wconv[i][:H], Wconv[i][H:], bconv[i],
                   gconv[i], bgconv[i])
        u_sum = gs_ucu(U, ucu_s, ucu_d, zU32)
        h_post = _post_combine(h_post, pub_sum, com_sum, Ecom, c_pub_dst,
                               c_com_dst, Wec[i], bec[i], g_ln_p, b_ln_p)
        h_user = _user_combine(h_user, u_sum, c_ucu_dst, g_ln_u, b_ln_u)

    W2p = jnp.pad(W2, ((0, 0), (0, 128 - W2.shape[1])))
    b2p = jnp.pad(b2, (0, 128 - b2.shape[0]))
    out = _classifier(h_user, W1, b1, gcls, bcls, W2p, b2p)
    return out[:, :W2.shape[1]]


# speedup vs baseline: 1.9292x; 1.9292x over previous
import functools

import jax
import jax.numpy as jnp
import numpy as np
from jax import lax
from jax.experimental import pallas as pl
from jax.experimental.pallas import tpu as pltpu
from jax.experimental.pallas import tpu_sc as plsc

NTILES = 16
NCORES = 2
KE = 128

f32 = jnp.float32
i32 = jnp.int32


def _rup(x, m):
    return -(-x // m) * m


def _mesh():
    return plsc.VectorSubcoreMesh(core_axis_name="c", subcore_axis_name="s")

_SC_PARAMS = dict(
    compiler_params=pltpu.CompilerParams(use_tc_tiling_on_sc=False))


def _pad_edges(src, dst, trash):
    E = src.shape[0]
    NB = -(-E // (NTILES * KE))
    pad = NTILES * KE * NB - E
    srcp = jnp.concatenate([src, jnp.zeros((pad,), i32)])
    dstp = jnp.concatenate([dst, jnp.full((pad,), trash, i32)])
    sd = jnp.stack([srcp.reshape(-1, KE), dstp.reshape(-1, KE)], axis=1)
    return sd, NB


def _seq_dst_pad(dst, trash):
    E = dst.shape[0]
    sh8 = _rup(-(-E // NTILES), 8)
    NB = -(-sh8 // KE)
    covered = np.zeros(E, bool)
    rowidx = np.zeros((NTILES, NB, KE), np.int32)
    newmask = np.zeros((NTILES, NB, KE), bool)
    for t in range(NTILES):
        for j in range(NB):
            s = min(t * sh8 + min(j * KE, sh8 - KE), E - KE)
            rows = np.arange(s, s + KE)
            rowidx[t, j] = rows
            newmask[t, j] = ~covered[rows]
            covered[rows] = True
    assert covered.all()
    dstp = jnp.where(jnp.asarray(newmask), dst[jnp.asarray(rowidx)], trash)
    return dstp.reshape(NTILES * NB, KE), NB, sh8, E


def _sc_gather_scatter(NB, C, CW, N, Np):
    CPC = C // NCORES
    rows = Np // NTILES

    @functools.partial(
        pl.kernel,
        mesh=_mesh(),
        out_type=jax.ShapeDtypeStruct((C, Np, CW), f32),
        scratch_types=[
            pltpu.VMEM((2, KE), i32), pltpu.VMEM((2, KE), i32),
            pltpu.VMEM((KE, CW), f32), pltpu.VMEM((KE, CW), f32),
            pltpu.VMEM_SHARED((Np, CW), f32),
            pltpu.SemaphoreType.DMA, pltpu.SemaphoreType.DMA,
            pltpu.SemaphoreType.DMA, pltpu.SemaphoreType.DMA,
        ],
        **_SC_PARAMS,
    )
    def k(table, sd, zeros, out, ib0, ib1, vb0, vb1, acc,
          si0, si1, sg0, sg1):
        cid = lax.axis_index("c")
        sid = lax.axis_index("s")
        r0 = sid * rows
        base = sid * NB
        for cc in range(CPC):
            chunk = cid * CPC + cc
            pltpu.sync_copy(zeros.at[pl.ds(r0, rows)], acc.at[pl.ds(r0, rows)])
            plsc.subcore_barrier()

            def idx_cp(j, ib, si):
                return pltpu.make_async_copy(sd.at[base + j], ib, si)

            def g_cp(ib, vb, sg, chunk=chunk):
                return pltpu.make_async_copy(
                    table.at[chunk].at[ib.at[0]], vb, sg)

            idx_cp(0, ib0, si0).start()
            idx_cp(0, ib0, si0).wait()
            g_cp(ib0, vb0, sg0).start()
            idx_cp(1, ib1, si1).start()

            def step(j, ci, cv, csi, csg, ni, nv, nsi, nsg):
                @pl.when(j + 1 < NB)
                def _():
                    idx_cp(j + 1, ni, nsi).wait()
                    g_cp(ni, nv, nsg).start()
                g_cp(ci, cv, csg).wait()
                pltpu.sync_copy(cv, acc.at[ci.at[1]], add=True)
                @pl.when(j + 2 < NB)
                def _():
                    idx_cp(j + 2, ci, csi).start()

            def body(j, carry):
                @pl.when(j % 2 == 0)
                def _():
                    step(j, ib0, vb0, si0, sg0, ib1, vb1, si1, sg1)
                @pl.when(j % 2 == 1)
                def _():
                    step(j, ib1, vb1, si1, sg1, ib0, vb0, si0, sg0)
                return carry

            lax.fori_loop(0, NB, body, 0)
            plsc.subcore_barrier()
            pltpu.sync_copy(acc.at[pl.ds(r0, rows)],
                            out.at[chunk].at[pl.ds(r0, rows)])

    return k


def _sc_seq_scatter(NB, sh8, E, C, CW, Np):
    CPC = C // NCORES
    rows = Np // NTILES

    @functools.partial(
        pl.kernel,
        mesh=_mesh(),
        out_type=jax.ShapeDtypeStruct((C, Np, CW), f32),
        scratch_types=[
            pltpu.VMEM((KE,), i32), pltpu.VMEM((KE,), i32),
            pltpu.VMEM((KE, CW), f32), pltpu.VMEM((KE, CW), f32),
            pltpu.VMEM_SHARED((Np, CW), f32),
            pltpu.SemaphoreType.DMA, pltpu.SemaphoreType.DMA,
            pltpu.SemaphoreType.DMA, pltpu.SemaphoreType.DMA,
        ],
        **_SC_PARAMS,
    )
    def k(ef, dstp, zeros, out, ib0, ib1, vb0, vb1, acc,
          si0, si1, sv0, sv1):
        cid = lax.axis_index("c")
        sid = lax.axis_index("s")
        r0 = sid * rows
        for cc in range(CPC):
            chunk = cid * CPC + cc
            pltpu.sync_copy(zeros.at[pl.ds(r0, rows)], acc.at[pl.ds(r0, rows)])
            plsc.subcore_barrier()

            def idx_cp(j, ib, si):
                return pltpu.make_async_copy(dstp.at[sid * NB + j], ib, si)

            def v_cp(j, vb, sv, chunk=chunk):
                off = jnp.minimum(
                    sid * sh8 + jnp.minimum(j * KE, sh8 - KE), E - KE)
                return pltpu.make_async_copy(
                    ef.at[chunk].at[pl.ds(off, KE)], vb, sv)

            idx_cp(0, ib0, si0).start()
            v_cp(0, vb0, sv0).start()

            def step(j, ci, cv, csi, csv, ni, nv, nsi, nsv):
                @pl.when(j + 1 < NB)
                def _():
                    idx_cp(j + 1, ni, nsi).start()
                    v_cp(j + 1, nv, nsv).start()
                idx_cp(j, ci, csi).wait()
                v_cp(j, cv, csv).wait()
                pltpu.sync_copy(cv, acc.at[ci], add=True)

            def body(j, carry):
                @pl.when(j % 2 == 0)
                def _():
                    step(j, ib0, vb0, si0, sv0, ib1, vb1, si1, sv1)
                @pl.when(j % 2 == 1)
                def _():
                    step(j, ib1, vb1, si1, sv1, ib0, vb0, si0, sv0)
                return carry

            lax.fori_loop(0, NB, body, 0)
            plsc.subcore_barrier()
            pltpu.sync_copy(acc.at[pl.ds(r0, rows)],
                            out.at[chunk].at[pl.ds(r0, rows)])

    return k


def _sc_counts(jobs):
    out_types = [jax.ShapeDtypeStruct((Np, 16), f32) for (_, Np) in jobs]
    accN = max(Np for (_, Np) in jobs)

    @functools.partial(
        pl.kernel,
        mesh=_mesh(),
        out_type=out_types,
        scratch_types=[
            pltpu.VMEM((2, KE), i32), pltpu.VMEM((2, KE), i32),
            pltpu.VMEM((KE, 16), f32),
            pltpu.VMEM_SHARED((accN, 16), f32),
            pltpu.SemaphoreType.DMA, pltpu.SemaphoreType.DMA,
        ],
        **_SC_PARAMS,
    )
    def k(*args):
        nj = len(jobs)
        dstps = args[:nj]
        ones, zeros = args[nj], args[nj + 1]
        outs = args[nj + 2:nj + 2 + nj]
        ib0, ib1, ones_v, acc, si0, si1 = args[nj + 2 + nj:]
        cid = lax.axis_index("c")
        sid = lax.axis_index("s")
        pltpu.sync_copy(ones, ones_v)
        for jj, (NB, Np) in enumerate(jobs):
            rows = Np // NTILES

            @pl.when(cid == (jj % 2))
            def _job(jj=jj, NB=NB, rows=rows):
                r0 = sid * rows
                base = sid * NB
                pltpu.sync_copy(zeros.at[pl.ds(r0, rows)],
                                acc.at[pl.ds(r0, rows)])
                plsc.subcore_barrier()

                def idx_cp(j, ib, si, jj=jj):
                    return pltpu.make_async_copy(
                        dstps[jj].at[base + j], ib, si)

                idx_cp(0, ib0, si0).start()

                def step(j, ci, csi, ni, nsi):
                    @pl.when(j + 1 < NB)
                    def _():
                        idx_cp(j + 1, ni, nsi).start()
                    idx_cp(j, ci, csi).wait()
                    pltpu.sync_copy(ones_v, acc.at[ci.at[1]], add=True)

                def body(j, carry):
                    @pl.when(j % 2 == 0)
                    def _():
                        step(j, ib0, si0, ib1, si1)
                    @pl.when(j % 2 == 1)
                    def _():
                        step(j, ib1, si1, ib0, si0)
                    return carry

                lax.fori_loop(0, NB, body, 0)
                plsc.subcore_barrier()
                pltpu.sync_copy(acc.at[pl.ds(r0, rows)],
                                outs[jj].at[pl.ds(r0, rows)])

    return k


def _ln(x, g, b, eps=1e-5):
    m = jnp.mean(x, -1, keepdims=True)
    v = jnp.mean((x - m) ** 2, -1, keepdims=True)
    return (x - m) * lax.rsqrt(v + eps) * g + b


def _row(BN, D):
    return pl.BlockSpec((BN, D), lambda i: (i, 0))


def _full(shape):
    return pl.BlockSpec(shape, lambda i: tuple(0 for _ in shape))


def _chunked(C, BN, CW):
    return pl.BlockSpec((C, BN, CW), lambda i: (0, i, 0))


def _cat(ref):
    C = ref.shape[0]
    return jnp.concatenate([ref[c] for c in range(C)], axis=-1)


def _store(o_ref, y, out_c):
    if out_c is None:
        o_ref[...] = y
    else:
        C, CW = out_c
        for c in range(C):
            o_ref[c] = y[:, c * CW:(c + 1) * CW]


def _proj(x, W, b, relu=False, out_c=None, BN=1000):
    N, Kd = x.shape
    H = W.shape[1]

    def f(x_ref, w_ref, b_ref, o_ref):
        y = jnp.dot(x_ref[...], w_ref[...], preferred_element_type=f32)
        y = y + b_ref[...]
        if relu:
            y = jnp.maximum(y, 0.0)
        _store(o_ref, y, out_c)

    if out_c is None:
        out_shape = jax.ShapeDtypeStruct((N, H), f32)
        out_spec = _row(BN, H)
    else:
        C, CW = out_c
        out_shape = jax.ShapeDtypeStruct((C, N, CW), f32)
        out_spec = _chunked(C, BN, CW)
    return pl.pallas_call(
        f, grid=(N // BN,),
        in_specs=[_row(BN, Kd), _full(W.shape), _full((1, H))],
        out_specs=out_spec, out_shape=out_shape,
    )(x, W, b.reshape(1, H))


def _ctx_final(S, cs, com, pub, N, H, BN=1000):
    def f(s_ref, cs_ref, com_ref, pub_ref, o_ref):
        cs1 = cs_ref[...][:, :1]
        y = _cat(s_ref) + 0.3 * _cat(com_ref)
        y = jnp.where(cs1 > 0, y / jnp.maximum(cs1, 1.0), y)
        o_ref[...] = y + 0.3 * _cat(pub_ref)

    return pl.pallas_call(
        f, grid=(N // BN,),
        in_specs=[_chunked(8, BN, 32), _row(BN, 16),
                  _chunked(8, BN, 32), _chunked(8, BN, 32)],
        out_specs=_row(BN, H),
        out_shape=jax.ShapeDtypeStruct((N, H), f32),
    )(S, cs, com, pub)


def _u_msg(h, ctx, Wa, Wb, b, g, bg, out_c=(8, 32), BN=1000):
    N, H = h.shape

    def f(h_ref, c_ref, wa_ref, wb_ref, b_ref, g_ref, bg_ref, o_ref):
        y = (jnp.dot(h_ref[...], wa_ref[...], preferred_element_type=f32)
             + jnp.dot(c_ref[...], wb_ref[...], preferred_element_type=f32)
             + b_ref[...])
        y = jnp.maximum(_ln(y, g_ref[...], bg_ref[...]), 0.0)
        _store(o_ref, y, out_c)

    C, CW = out_c
    return pl.pallas_call(
        f, grid=(N // BN,),
        in_specs=[_row(BN, H), _row(BN, H), _full(Wa.shape), _full(Wb.shape),
                  _full((1, H)), _full((1, H)), _full((1, H))],
        out_specs=_chunked(C, BN, CW),
        out_shape=jax.ShapeDtypeStruct((C, N, CW), f32),
    )(h, ctx, Wa, Wb, b.reshape(1, H), g.reshape(1, H), bg.reshape(1, H))


def _post_combine(h, pub_sum, com_sum, Ecom, cp, cc, Wec, bec, g, b, BN=2000):
    N, H = h.shape

    def f(h_ref, ps_ref, cs_ref, e_ref, cp_ref, cc_ref, w_ref, be_ref,
          g_ref, b_ref, o_ref):
        cp1 = jnp.maximum(cp_ref[...][:, :1], 1.0)
        ccr = cc_ref[...][:, :1]
        cc1 = jnp.maximum(ccr, 1.0)
        ep = (jnp.dot(_cat(e_ref), w_ref[...], preferred_element_type=f32)
              + ccr * be_ref[...])
        aggc = (0.7 * _cat(cs_ref) + 0.3 * ep) / cc1
        aggp = _cat(ps_ref) / cp1
        o_ref[...] = _ln(h_ref[...] + 0.5 * (aggp + aggc), g_ref[...], b_ref[...])

    return pl.pallas_call(
        f, grid=(N // BN,),
        in_specs=[_row(BN, H), _chunked(2, BN, 128), _chunked(2, BN, 128),
                  _chunked(2, BN, 128), _row(BN, 16), _row(BN, 16),
                  _full(Wec.shape), _full((1, H)), _full((1, H)),
                  _full((1, H))],
        out_specs=_row(BN, H),
        out_shape=jax.ShapeDtypeStruct((N, H), f32),
    )(h, pub_sum, com_sum, Ecom, cp, cc, Wec, bec.reshape(1, H),
      g.reshape(1, H), b.reshape(1, H))


def _user_combine(h, u_sum, cu, g, b, BN=2000):
    N, H = h.shape

    def f(h_ref, u_ref, c_ref, g_ref, b_ref, o_ref):
        c1 = jnp.maximum(c_ref[...][:, :1], 1.0)
        o_ref[...] = _ln(h_ref[...] + _cat(u_ref) / c1, g_ref[...], b_ref[...])

    return pl.pallas_call(
        f, grid=(N // BN,),
        in_specs=[_row(BN, H), _chunked(8, BN, 32), _row(BN, 16),
                  _full((1, H)), _full((1, H))],
        out_specs=_row(BN, H),
        out_shape=jax.ShapeDtypeStruct((N, H), f32),
    )(h, u_sum, cu, g.reshape(1, H), b.reshape(1, H))


def _classifier(h, W1, b1, g, b, W2p, b2p, BN=2000):
    N, H = h.shape
    Ho = W2p.shape[1]

    def f(h_ref, w1_ref, b1_ref, g_ref, b_ref, w2_ref, b2_ref, o_ref):
        y = jnp.dot(h_ref[...], w1_ref[...], preferred_element_type=f32) + b1_ref[...]
        y = jnp.maximum(_ln(y, g_ref[...], b_ref[...]), 0.0)
        o_ref[...] = jnp.dot(y, w2_ref[...], preferred_element_type=f32) + b2_ref[...]

    return pl.pallas_call(
        f, grid=(N // BN,),
        in_specs=[_row(BN, H), _full(W1.shape), _full((1, H)), _full((1, H)),
                  _full((1, H)), _full(W2p.shape), _full((1, Ho))],
        out_specs=_row(BN, Ho),
        out_shape=jax.ShapeDtypeStruct((N, Ho), f32),
    )(h, W1, b1.reshape(1, H), g.reshape(1, H), b.reshape(1, H), W2p,
      b2p.reshape(1, Ho))


def kernel(user_x, post_x, ef_com, ef_ucu, edge_pub, edge_com, edge_ucu,
           Wu, bu, Wp, bp, Wepc, bepc, Wepu, bepu, Wuce, buce,
           Wpub, bpub, Wcom, bcom, Wconv, bconv, gconv, bgconv, Wec, bec,
           g_ln_u, b_ln_u, g_ln_p, b_ln_p, W1, b1, gcls, bcls, W2, b2):
    NU, NPOST, H = user_x.shape[0], post_x.shape[0], Wu.shape[1]
    NUp, NPp = _rup(NU + 1, 128), _rup(NPOST + 1, 128)

    zU32 = jnp.zeros((NUp, 32), f32)
    zP128 = jnp.zeros((NPp, 128), f32)
    zU16 = jnp.zeros((NUp, 16), f32)
    ones16 = jnp.ones((KE, 16), f32)

    ucu_sd, NB_ucu = _pad_edges(edge_ucu[0], edge_ucu[1], NU)
    com_sd, NB_com = _pad_edges(edge_com[0], edge_com[1], NPOST)
    pub_sd, NB_pub = _pad_edges(edge_pub[0], edge_pub[1], NPOST)
    comR_sd, _ = _pad_edges(edge_com[1], edge_com[0], NU)
    pubR_sd, _ = _pad_edges(edge_pub[1], edge_pub[0], NU)
    com_src_sd, _ = _pad_edges(edge_com[0], edge_com[0], NU)
    ucu_seq_d, NBs_ucu, sh_ucu, EU = _seq_dst_pad(edge_ucu[0], NU)
    com_seq_d, NBs_com, sh_com, EC = _seq_dst_pad(edge_com[1], NPOST)

    cnt_jobs = [(NB_ucu, NUp), (NB_com, NUp), (NB_com, NPp), (NB_pub, NPp)]
    c_ucu_dst, c_com_src, c_com_dst, c_pub_dst = _sc_counts(cnt_jobs)(
        ucu_sd, com_src_sd, com_sd, pub_sd, ones16, zU16)

    h_user = _proj(user_x, Wu, bu)
    h_post = _proj(post_x, Wp, bp, BN=2000)
    post_ctx = _proj(post_x, Wuce, buce, relu=True, out_c=(8, 32), BN=2000)
    P_ucu = _proj(ef_ucu, Wepu, bepu, out_c=(8, 32))
    P_com = _proj(ef_com, Wepc, bepc, out_c=(2, 128))

    S_ucu = _sc_seq_scatter(NBs_ucu, sh_ucu, EU, 8, 32, NUp)(
        P_ucu, ucu_seq_d, zU32)
    Ecom = _sc_seq_scatter(NBs_com, sh_com, EC, 2, 128, NPp)(
        P_com, com_seq_d, zP128)

    ctx_com = _sc_gather_scatter(NB_com, 8, 32, NU, NUp)(
        post_ctx, comR_sd, zU32)
    ctx_pub = _sc_gather_scatter(NB_pub, 8, 32, NU, NUp)(
        post_ctx, pubR_sd, zU32)
    ctx = _ctx_final(S_ucu, c_com_src, ctx_com, ctx_pub, NU, H)

    gs_pub = _sc_gather_scatter(NB_pub, 2, 128, NPOST, NPp)
    gs_com = _sc_gather_scatter(NB_com, 2, 128, NPOST, NPp)
    gs_ucu = _sc_gather_scatter(NB_ucu, 8, 32, NU, NUp)

    L = Wpub.shape[0]
    for i in range(L):
        t_pub = _proj(h_user, Wpub[i], bpub[i], out_c=(2, 128))
        t_com = _proj(h_user, Wcom[i], bcom[i], out_c=(2, 128))
        pub_sum = gs_pub(t_pub, pub_sd, zP128)
        com_sum = gs_com(t_com, com_sd, zP128)
        U = _u_msg(h_user, ctx, Wconv[i][:H], Wconv[i][H:], bconv[i],
                   gconv[i], bgconv[i])
        u_sum = gs_ucu(U, ucu_sd, zU32)
        h_post = _post_combine(h_post, pub_sum, com_sum, Ecom, c_pub_dst,
                               c_com_dst, Wec[i], bec[i], g_ln_p, b_ln_p)
        h_user = _user_combine(h_user, u_sum, c_ucu_dst, g_ln_u, b_ln_u)

    W2p = jnp.pad(W2, ((0, 0), (0, 128 - W2.shape[1])))
    b2p = jnp.pad(b2, (0, 128 - b2.shape[0]))
    out = _classifier(h_user, W1, b1, gcls, bcls, W2p, b2p)
    return out[:, :W2.shape[1]]
